# Initial kernel scaffold; baseline (speedup 1.0000x reference)
import jax
import jax.numpy as jnp
from jax.experimental import pallas as pl


def kernel(x, Wq_i, Wk_i, Wv_i, b_i, Wq_h, Wk_h, Wv_h, b_h, ln_g, ln_b):
    raise NotImplementedError("write your pallas kernel here")


# speedup vs baseline: 2.4234x; 2.4234x over previous
import jax
import jax.numpy as jnp
from jax.experimental import pallas as pl
from jax.experimental.pallas import tpu as pltpu

_B, _T, _N, _D, _L = 16, 12, 325, 64, 2
_SCALE = 1.0 / (_D ** 0.5)


def _garnn_kernel(x_ref, w_ref, p_ref, out_ref, hid_ref, ai_ref, ah_ref,
                  h_scr, o_scr):
    t = pl.program_id(1)
    l = pl.program_id(2)

    x_in = jnp.where(l == 0, x_ref[0, 0], o_scr[...])

    w = w_ref[0]
    p = p_ref[0]
    qkv = jnp.dot(x_in, w, preferred_element_type=jnp.float32)

    def attn_branch(q, k, v, bias):
        s = jax.lax.dot_general(q, k, (((1,), (1,)), ((), ())),
                                preferred_element_type=jnp.float32) * _SCALE
        a = jax.nn.softmax(s, axis=-1)
        o = jnp.dot(a, v, preferred_element_type=jnp.float32) + bias
        return o, a

    oi, ai = attn_branch(qkv[:, 0:_D], qkv[:, _D:2 * _D],
                         qkv[:, 2 * _D:5 * _D], p[:, 0:3 * _D])
    oh, ah = attn_branch(qkv[:, 5 * _D:6 * _D], qkv[:, 6 * _D:7 * _D],
                         qkv[:, 7 * _D:10 * _D], p[:, 3 * _D:6 * _D])

    h_prev = jnp.where(t == 0, jnp.zeros((_N, _D), jnp.float32), h_scr[l])

    r = jax.nn.sigmoid(oi[:, 0:_D] + oh[:, 0:_D])
    z = jax.nn.sigmoid(oi[:, _D:2 * _D] + oh[:, _D:2 * _D])
    n = jnp.tanh(oi[:, 2 * _D:3 * _D] + r * oh[:, 2 * _D:3 * _D])
    o = n + z * (h_prev - n)

    m = jnp.mean(o, axis=-1, keepdims=True)
    v = jnp.mean((o - m) * (o - m), axis=-1, keepdims=True)
    o = (o - m) / jnp.sqrt(v + 1e-5) * p[:, 6 * _D:7 * _D] + p[:, 7 * _D:8 * _D]

    h_scr[l] = o
    o_scr[...] = o
    out_ref[0, 0] = o
    hid_ref[0, 0] = o
    ai_ref[0, 0, 0] = ai
    ah_ref[0, 0, 0] = ah


def _run(x, wcat, pcat):
    grid = (_B, _T, _L)
    out_shape = (
        jax.ShapeDtypeStruct((_B, _T, _N, _D), jnp.float32),
        jax.ShapeDtypeStruct((_B, _L, _N, _D), jnp.float32),
        jax.ShapeDtypeStruct((_B, _T, _L, _N, _N), jnp.float32),
        jax.ShapeDtypeStruct((_B, _T, _L, _N, _N), jnp.float32),
    )
    in_specs = [
        pl.BlockSpec((1, 1, _N, _D), lambda b, t, l: (b, t, 0, 0)),
        pl.BlockSpec((1, _D, 10 * _D), lambda b, t, l: (l, 0, 0)),
        pl.BlockSpec((1, 1, 8 * _D), lambda b, t, l: (l, 0, 0)),
    ]
    out_specs = (
        pl.BlockSpec((1, 1, _N, _D), lambda b, t, l: (b, t, 0, 0)),
        pl.BlockSpec((1, 1, _N, _D), lambda b, t, l: (b, l, 0, 0)),
        pl.BlockSpec((1, 1, 1, _N, _N), lambda b, t, l: (b, t, l, 0, 0)),
        pl.BlockSpec((1, 1, 1, _N, _N), lambda b, t, l: (b, t, l, 0, 0)),
    )
    return pl.pallas_call(
        _garnn_kernel,
        grid=grid,
        in_specs=in_specs,
        out_specs=out_specs,
        out_shape=out_shape,
        scratch_shapes=[
            pltpu.VMEM((_L, _N, _D), jnp.float32),
            pltpu.VMEM((_N, _D), jnp.float32),
        ],
        compiler_params=pltpu.CompilerParams(
            dimension_semantics=("parallel", "arbitrary", "arbitrary"),
        ),
    )(x, wcat, pcat)


def kernel(x, Wq_i, Wk_i, Wv_i, b_i, Wq_h, Wk_h, Wv_h, b_h, ln_g, ln_b):
    wcat = jnp.concatenate([Wq_i, Wk_i, Wv_i, Wq_h, Wk_h, Wv_h], axis=-1)
    pcat = jnp.concatenate([b_i, b_h, ln_g, ln_b], axis=-1)[:, None, :]
    return _run(x, wcat, pcat)
